# baseline (speedup 1.0000x reference)
import functools

import jax
import jax.numpy as jnp
import numpy as np
from jax.experimental import pallas as pl
from jax.experimental.pallas import tpu as pltpu

RES = 5
SIGMA = 0.1
M_SLICES = 50

_E_BLOCK = 12800


def _img_body(pd_ref, out_ref):
    i = pl.program_id(0)

    @pl.when(i == 0)
    def _():
        out_ref[...] = jnp.zeros_like(out_ref)

    pd = pd_ref[...]
    births = pd[:, 0]
    pers = pd[:, 1] - pd[:, 0]
    centers = (jax.lax.broadcasted_iota(jnp.int32, (1, RES), 1).astype(jnp.float32) + 0.5) / RES
    w = jnp.clip(pers, 0.0, 1.0)
    gb = jnp.exp(-((centers - births[:, None]) ** 2) / (2.0 * SIGMA**2))
    gp = jnp.exp(-((centers - pers[:, None]) ** 2) / (2.0 * SIGMA**2))
    img = jnp.dot((w[:, None] * gb).T, gp, preferred_element_type=jnp.float32)
    xd = jnp.sum(jnp.abs(pd[:, 1] - pd[:, 0]))
    acc = jnp.pad(img, ((0, 3), (0, 123)))
    row = jax.lax.broadcasted_iota(jnp.int32, (8, 128), 0)
    col = jax.lax.broadcasted_iota(jnp.int32, (8, 128), 1)
    acc = acc + jnp.where((row == 7) & (col == 127), xd, 0.0)
    out_ref[...] += acc


def _img_pallas(pd):
    ne = pd.shape[0]
    grid = ne // _E_BLOCK
    out = pl.pallas_call(
        _img_body,
        grid=(grid,),
        in_specs=[pl.BlockSpec((_E_BLOCK, 2), lambda i: (i, 0))],
        out_specs=pl.BlockSpec((8, 128), lambda i: (0, 0)),
        out_shape=jax.ShapeDtypeStruct((8, 128), jnp.float32),
    )(pd)
    img = out[:RES, :RES] / (2.0 * np.pi * SIGMA**2)
    xd_sum = out[7, 127]
    return img.reshape(-1), xd_sum


def _prelu(x):
    return jnp.where(x > 0, x, 0.1 * x)


def kernel(x0, edge_index0, PD, W1, b1, W2, b2, W4, b4, W3, b3, W5, b5, W6, b6):
    n = x0.shape[0]
    src = edge_index0[0]
    dst = edge_index0[1]

    def gin(x, W, b, relu):
        agg = jax.ops.segment_sum(x[src], dst, num_segments=n)
        h = (x + agg) @ W + b
        return jnp.maximum(h, 0.0) if relu else h

    h = gin(x0, W1, b1, True)
    h = gin(h, W2, b2, True)
    h = gin(h, W4, b4, True)
    h = gin(h, W3, b3, False)

    rs = src[:-n]
    rd = dst[:-n]
    e = jnp.concatenate([h[rs], h[rd]], axis=-1) @ W5 + b5
    e = _prelu(e)
    pd = e @ W6 + b6

    def dproj(p):
        m = (p[:, 0] + p[:, 1]) * 0.5
        return jnp.stack([m, m], axis=-1)

    Xa = jnp.concatenate([pd, dproj(PD)], axis=0)
    Ya = jnp.concatenate([PD, dproj(pd)], axis=0)
    th = jnp.linspace(-np.pi / 2, np.pi / 2, M_SLICES, endpoint=False).astype(jnp.float32)
    dirs = jnp.stack([jnp.cos(th), jnp.sin(th)], axis=0)
    sx = jnp.sort(Xa @ dirs, axis=0)
    sy = jnp.sort(Ya @ dirs, axis=0)
    loss0 = jnp.mean(jnp.abs(sx - sy))
    loss_xy = loss0

    img, xd_sum = _img_pallas(pd)
    loss_xd = (xd_sum / pd.shape[0]) / jnp.sqrt(2.0)
    loss_yd = jnp.mean(jnp.abs(PD[:, 1] - PD[:, 0])) / jnp.sqrt(2.0)
    return pd, img, loss0, loss_xy, loss_xd, loss_yd


# speedup vs baseline: 5.8198x; 5.8198x over previous
import functools

import jax
import jax.numpy as jnp
import numpy as np
from jax import lax
from jax.experimental import pallas as pl
from jax.experimental.pallas import tpu as pltpu
from jax.experimental.pallas import tpu_sc as plsc

RES = 5
SIGMA = 0.1
M_SLICES = 50

_E_BLOCK = 12800
_NE = 1600000
_NQ = 1024
_K = 32768
_CH = 4000
_NCH = _NE // _CH
_NSLOT = 64


def _img_body(pd_ref, out_ref, mx_ref):
    i = pl.program_id(0)

    @pl.when(i == 0)
    def _():
        out_ref[...] = jnp.zeros_like(out_ref)
        mx_ref[...] = jnp.zeros_like(mx_ref)

    pd = pd_ref[...]
    births = pd[:, 0]
    pers = pd[:, 1] - pd[:, 0]
    centers = (jax.lax.broadcasted_iota(jnp.int32, (1, RES), 1).astype(jnp.float32) + 0.5) / RES
    w = jnp.clip(pers, 0.0, 1.0)
    gb = jnp.exp(-((centers - births[:, None]) ** 2) / (2.0 * SIGMA**2))
    gp = jnp.exp(-((centers - pers[:, None]) ** 2) / (2.0 * SIGMA**2))
    img = jnp.dot((w[:, None] * gb).T, gp, preferred_element_type=jnp.float32)
    xd = jnp.sum(jnp.abs(pd[:, 1] - pd[:, 0]))
    acc = jnp.pad(img, ((0, 3), (0, 123)))
    row = jax.lax.broadcasted_iota(jnp.int32, (8, 128), 0)
    col = jax.lax.broadcasted_iota(jnp.int32, (8, 128), 1)
    acc = acc + jnp.where((row == 7) & (col == 127), xd, 0.0)
    out_ref[...] += acc
    r = jnp.max(jnp.abs(pd[:, 0]) + jnp.abs(pd[:, 1]))
    mx_ref[...] = jnp.maximum(mx_ref[...], jnp.full((8, 128), r))


def _img_pallas(pd):
    ne = pd.shape[0]
    grid = ne // _E_BLOCK
    out, mx = pl.pallas_call(
        _img_body,
        grid=(grid,),
        in_specs=[pl.BlockSpec((_E_BLOCK, 2), lambda i: (i, 0))],
        out_specs=[pl.BlockSpec((8, 128), lambda i: (0, 0)),
                   pl.BlockSpec((8, 128), lambda i: (0, 0))],
        out_shape=[jax.ShapeDtypeStruct((8, 128), jnp.float32),
                   jax.ShapeDtypeStruct((8, 128), jnp.float32)],
    )(pd)
    img = out[:RES, :RES] / (2.0 * np.pi * SIGMA**2)
    xd_sum = out[7, 127]
    maxr = mx[0, 0]
    return img.reshape(-1), xd_sum, maxr


def _sw_body(pd0_h, pd1_h, q0_h, q1_h, ct_h, st_h, lo_h, invw_h, out_h,
             hist0, hist1, buf0, buf1, qb0, qb1,
             c0v, s0v, c1v, s1v, lov, invwv, res):
    wid = lax.axis_index("s") * 2 + lax.axis_index("c")
    slot0 = wid
    slot1 = wid + 32

    pltpu.sync_copy(ct_h.at[slot0], c0v)
    pltpu.sync_copy(st_h.at[slot0], s0v)
    pltpu.sync_copy(ct_h.at[slot1], c1v)
    pltpu.sync_copy(st_h.at[slot1], s1v)
    pltpu.sync_copy(lo_h, lov)
    pltpu.sync_copy(invw_h, invwv)
    pltpu.sync_copy(q0_h, qb0)
    pltpu.sync_copy(q1_h, qb1)

    c0 = c0v[...]
    s0 = s0v[...]
    c1 = c1v[...]
    s1 = s1v[...]
    lo = lov[...]
    invw = invwv[...]

    zeros16 = jnp.zeros((16,), jnp.int32)

    def zb(i, _):
        hist0[pl.ds(i * 16, 16)] = zeros16
        hist1[pl.ds(i * 16, 16)] = zeros16
        return 0

    lax.fori_loop(0, _K // 16, zb, 0)

    ones = jnp.ones((16,), jnp.int32)
    nones = jnp.full((16,), -1, jnp.int32)
    kmax = jnp.full((16,), float(_K - 1), jnp.float32)
    zf = jnp.zeros((16,), jnp.float32)

    def binof(v):
        t = (v - lo) * invw
        t = jnp.minimum(jnp.maximum(t, zf), kmax)
        return t.astype(jnp.int32)

    def scat(p0, p1):
        m = (p0 + p1) * 0.5
        plsc.addupdate_scatter(hist0, [binof(p0 * c0 + p1 * s0)], ones)
        plsc.addupdate_scatter(hist0, [binof(m * (c0 + s0))], nones)
        plsc.addupdate_scatter(hist1, [binof(p0 * c1 + p1 * s1)], ones)
        plsc.addupdate_scatter(hist1, [binof(m * (c1 + s1))], nones)

    def chunk_body(ci, _):
        pltpu.sync_copy(pd0_h.at[pl.ds(ci * _CH, _CH)], buf0)
        pltpu.sync_copy(pd1_h.at[pl.ds(ci * _CH, _CH)], buf1)

        def vbody(vi, _):
            scat(buf0[pl.ds(vi * 16, 16)], buf1[pl.ds(vi * 16, 16)])
            return 0

        lax.fori_loop(0, _CH // 16, vbody, 0)
        return 0

    lax.fori_loop(0, _NCH, chunk_body, 0)

    def qbody(vi, _):
        p0 = qb0[pl.ds(vi * 16, 16)]
        p1 = qb1[pl.ds(vi * 16, 16)]
        m = (p0 + p1) * 0.5
        plsc.addupdate_scatter(hist0, [binof(m * (c0 + s0))], ones)
        plsc.addupdate_scatter(hist0, [binof(p0 * c0 + p1 * s0)], nones)
        plsc.addupdate_scatter(hist1, [binof(m * (c1 + s1))], ones)
        plsc.addupdate_scatter(hist1, [binof(p0 * c1 + p1 * s1)], nones)
        return 0

    lax.fori_loop(0, _NQ // 16, qbody, 0)

    def absum(hist):
        def body(i, carry_acc):
            carry, acc = carry_acc
            v = hist[pl.ds(i * 16, 16)]
            cum = plsc.cumsum(v) + carry
            acc = acc + jnp.sum(jnp.abs(cum).astype(jnp.float32))
            carry = carry + jnp.sum(v)
            return (carry, acc)

        _, acc = lax.fori_loop(0, _K // 16, body, (jnp.int32(0), jnp.float32(0.0)))
        return acc

    acc0 = absum(hist0)
    acc1 = absum(hist1)
    acc1 = jnp.where(wid < 18, acc1, 0.0)

    res[...] = jnp.broadcast_to(acc0, (16,))
    pltpu.sync_copy(res, out_h.at[slot0])
    res[...] = jnp.broadcast_to(acc1, (16,))
    pltpu.sync_copy(res, out_h.at[slot1])


@functools.partial(jax.jit, static_argnames=())
def _sw_pallas(pd0, pd1, q0, q1, ct, st, lo, invw):
    mesh = plsc.VectorSubcoreMesh(core_axis_name="c", subcore_axis_name="s")
    f = functools.partial(
        pl.kernel,
        mesh=mesh,
        compiler_params=pltpu.CompilerParams(needs_layout_passes=False),
        out_type=jax.ShapeDtypeStruct((_NSLOT, 16), jnp.float32),
        scratch_types=[
            pltpu.VMEM((_K,), jnp.int32),
            pltpu.VMEM((_K,), jnp.int32),
            pltpu.VMEM((_CH,), jnp.float32),
            pltpu.VMEM((_CH,), jnp.float32),
            pltpu.VMEM((_NQ,), jnp.float32),
            pltpu.VMEM((_NQ,), jnp.float32),
            pltpu.VMEM((16,), jnp.float32),
            pltpu.VMEM((16,), jnp.float32),
            pltpu.VMEM((16,), jnp.float32),
            pltpu.VMEM((16,), jnp.float32),
            pltpu.VMEM((16,), jnp.float32),
            pltpu.VMEM((16,), jnp.float32),
            pltpu.VMEM((16,), jnp.float32),
        ],
    )(_sw_body)
    return f(pd0, pd1, q0, q1, ct, st, lo, invw)


def _prelu(x):
    return jnp.where(x > 0, x, 0.1 * x)


def kernel(x0, edge_index0, PD, W1, b1, W2, b2, W4, b4, W3, b3, W5, b5, W6, b6):
    n = x0.shape[0]
    src = edge_index0[0]
    dst = edge_index0[1]

    def gin(x, W, b, relu):
        agg = jax.ops.segment_sum(x[src], dst, num_segments=n)
        h = (x + agg) @ W + b
        return jnp.maximum(h, 0.0) if relu else h

    h = gin(x0, W1, b1, True)
    h = gin(h, W2, b2, True)
    h = gin(h, W4, b4, True)
    h = gin(h, W3, b3, False)

    rs = src[:-n]
    rd = dst[:-n]
    e = jnp.concatenate([h[rs], h[rd]], axis=-1) @ W5 + b5
    e = _prelu(e)
    pd = e @ W6 + b6

    img, xd_sum, maxr = _img_pallas(pd)
    loss_xd = (xd_sum / pd.shape[0]) / jnp.sqrt(2.0)
    loss_yd = jnp.mean(jnp.abs(PD[:, 1] - PD[:, 0])) / jnp.sqrt(2.0)

    th = np.linspace(-np.pi / 2, np.pi / 2, M_SLICES, endpoint=False)
    cth = np.cos(th).astype(np.float32)
    sth = np.sin(th).astype(np.float32)
    ct = np.zeros((_NSLOT, 16), np.float32)
    st = np.zeros((_NSLOT, 16), np.float32)
    ct[:M_SLICES, :] = cth[:, None]
    st[:M_SLICES, :] = sth[:, None]
    ct = jnp.asarray(ct)
    st = jnp.asarray(st)

    maxr = jnp.maximum(maxr, 2.0) + 1.0
    lo_s = -maxr
    w_s = (2.0 * maxr) / _K
    invw_s = 1.0 / w_s
    lo_v = jnp.full((16,), lo_s, jnp.float32)
    invw_v = jnp.full((16,), invw_s, jnp.float32)

    pd0 = pd[:, 0]
    pd1 = pd[:, 1]
    q0 = PD[:, 0]
    q1 = PD[:, 1]

    out = _sw_pallas(pd0, pd1, q0, q1, ct, st, lo_v, invw_v)
    n_tot = _NE + _NQ
    loss0 = jnp.sum(out[:, 0]) * w_s / (n_tot * M_SLICES)
    loss_xy = loss0

    return pd, img, loss0, loss_xy, loss_xd, loss_yd


# speedup vs baseline: 8.2644x; 1.4201x over previous
import functools

import jax
import jax.numpy as jnp
import numpy as np
from jax import lax
from jax.experimental import pallas as pl
from jax.experimental.pallas import tpu as pltpu
from jax.experimental.pallas import tpu_sc as plsc

RES = 5
SIGMA = 0.1
M_SLICES = 50

_NE = 1600000
_NA = 1700000
_NN = 100000
_HID = 32
_NQ = 1024
_K = 32768
_CH = 4000
_NCH = _NE // _CH
_NSLOT = 64

_GC = 400
_PW = 128
_E_BLOCK = 6400
_SC_MESH = dict(core_axis_name="c", subcore_axis_name="s")
_SC_PARAMS = None


def _wid():
    return lax.axis_index("s") * 2 + lax.axis_index("c")


def _g1_body(t_h, src_h, g_h, idxb, buf, sem):
    wid = _wid()
    nfull = (_NA // _GC) // 32
    nrem = (_NA // _GC) - nfull * 32
    nch = jnp.where(wid < nrem, nfull + 1, nfull)

    def chunk(j, _):
        c = j * 32 + wid
        off = c * _GC
        pltpu.sync_copy(src_h.at[pl.ds(off, _GC)], idxb)
        pltpu.async_copy(t_h.at[idxb], buf, sem).wait()
        pltpu.sync_copy(buf, g_h.at[pl.ds(off, _GC)])
        return 0

    lax.fori_loop(0, nch, chunk, 0)


def _g1(table, src):
    f = pl.kernel(
        _g1_body,
        mesh=plsc.VectorSubcoreMesh(**_SC_MESH),
        compiler_params=pltpu.CompilerParams(needs_layout_passes=False),
        out_type=jax.ShapeDtypeStruct((_NA, _PW), jnp.float32),
        scratch_types=[
            pltpu.VMEM((_GC,), jnp.int32),
            pltpu.VMEM((_GC, _PW), jnp.float32),
            pltpu.SemaphoreType.DMA,
        ],
    )
    return f(table, src)


def _g2_body(a_h, b_h, rs_h, rd_h, ea_h, eb_h, idx1, idx2, bufa, bufb, sem1, sem2):
    wid = _wid()
    base = wid * (_NE // 32)

    def chunk(j, _):
        off = base + j * _GC
        pltpu.sync_copy(rs_h.at[pl.ds(off, _GC)], idx1)
        pltpu.sync_copy(rd_h.at[pl.ds(off, _GC)], idx2)
        cp1 = pltpu.async_copy(a_h.at[idx1], bufa, sem1)
        cp2 = pltpu.async_copy(b_h.at[idx2], bufb, sem2)
        cp1.wait()
        cp2.wait()
        pltpu.sync_copy(bufa, ea_h.at[pl.ds(off, _GC)])
        pltpu.sync_copy(bufb, eb_h.at[pl.ds(off, _GC)])
        return 0

    lax.fori_loop(0, _NE // 32 // _GC, chunk, 0)


def _g2(a, b, rs, rd):
    f = pl.kernel(
        _g2_body,
        mesh=plsc.VectorSubcoreMesh(**_SC_MESH),
        compiler_params=pltpu.CompilerParams(needs_layout_passes=False),
        out_type=[jax.ShapeDtypeStruct((_NE, _PW), jnp.float32),
                  jax.ShapeDtypeStruct((_NE, _PW), jnp.float32)],
        scratch_types=[
            pltpu.VMEM((_GC,), jnp.int32),
            pltpu.VMEM((_GC,), jnp.int32),
            pltpu.VMEM((_GC, _PW), jnp.float32),
            pltpu.VMEM((_GC, _PW), jnp.float32),
            pltpu.SemaphoreType.DMA,
            pltpu.SemaphoreType.DMA,
        ],
    )
    return f(a, b, rs, rd)


def _pd_body(ea_ref, eb_ref, w6_ref, b6_ref, pd_ref, p0_ref, p1_ref, sums_ref, mx_ref):
    i = pl.program_id(0)

    @pl.when(i == 0)
    def _():
        sums_ref[...] = jnp.zeros_like(sums_ref)
        mx_ref[...] = jnp.zeros_like(mx_ref)

    f = ea_ref[:, :_HID] + eb_ref[:, :_HID]
    f = jnp.where(f > 0, f, 0.1 * f)
    pdb = jnp.dot(f, w6_ref[...], preferred_element_type=jnp.float32) + b6_ref[...]
    pd_ref[...] = pdb
    p0_ref[...] = pdb[:, 0:1]
    p1_ref[...] = pdb[:, 1:2]

    births = pdb[:, 0]
    pers = pdb[:, 1] - pdb[:, 0]
    centers = (jax.lax.broadcasted_iota(jnp.int32, (1, RES), 1).astype(jnp.float32) + 0.5) / RES
    w = jnp.clip(pers, 0.0, 1.0)
    gb = jnp.exp(-((centers - births[:, None]) ** 2) / (2.0 * SIGMA**2))
    gp = jnp.exp(-((centers - pers[:, None]) ** 2) / (2.0 * SIGMA**2))
    img = jnp.dot((w[:, None] * gb).T, gp, preferred_element_type=jnp.float32)
    xd = jnp.sum(jnp.abs(pers))
    acc = jnp.pad(img, ((0, 3), (0, 123)))
    row = jax.lax.broadcasted_iota(jnp.int32, (8, 128), 0)
    col = jax.lax.broadcasted_iota(jnp.int32, (8, 128), 1)
    acc = acc + jnp.where((row == 7) & (col == 127), xd, 0.0)
    sums_ref[...] += acc
    r = jnp.max(jnp.abs(pdb[:, 0]) + jnp.abs(pdb[:, 1]))
    mx_ref[...] = jnp.maximum(mx_ref[...], jnp.full((8, 128), r))


def _pd_pallas(ea, eb, w6, b6):
    grid = _NE // _E_BLOCK
    pd, p0, p1, sums, mx = pl.pallas_call(
        _pd_body,
        grid=(grid,),
        in_specs=[
            pl.BlockSpec((_E_BLOCK, _PW), lambda i: (i, 0)),
            pl.BlockSpec((_E_BLOCK, _PW), lambda i: (i, 0)),
            pl.BlockSpec((_HID, 2), lambda i: (0, 0)),
            pl.BlockSpec((1, 2), lambda i: (0, 0)),
        ],
        out_specs=[
            pl.BlockSpec((_E_BLOCK, 2), lambda i: (i, 0)),
            pl.BlockSpec((_E_BLOCK, 1), lambda i: (i, 0)),
            pl.BlockSpec((_E_BLOCK, 1), lambda i: (i, 0)),
            pl.BlockSpec((8, 128), lambda i: (0, 0)),
            pl.BlockSpec((8, 128), lambda i: (0, 0)),
        ],
        out_shape=[
            jax.ShapeDtypeStruct((_NE, 2), jnp.float32),
            jax.ShapeDtypeStruct((_NE, 1), jnp.float32),
            jax.ShapeDtypeStruct((_NE, 1), jnp.float32),
            jax.ShapeDtypeStruct((8, 128), jnp.float32),
            jax.ShapeDtypeStruct((8, 128), jnp.float32),
        ],
    )(ea, eb, w6, b6)
    img = sums[:RES, :RES] / (2.0 * np.pi * SIGMA**2)
    xd_sum = sums[7, 127]
    maxr = mx[0, 0]
    return pd, p0.reshape(-1), p1.reshape(-1), img.reshape(-1), xd_sum, maxr


def _sw_body(pd0_h, pd1_h, q0_h, q1_h, ct_h, st_h, lo_h, invw_h, out_h,
             hist0, hist1, buf0, buf1, qb0, qb1,
             c0v, s0v, c1v, s1v, lov, invwv, res):
    wid = _wid()
    slot0 = wid
    slot1 = wid + 32

    pltpu.sync_copy(ct_h.at[slot0], c0v)
    pltpu.sync_copy(st_h.at[slot0], s0v)
    pltpu.sync_copy(ct_h.at[slot1], c1v)
    pltpu.sync_copy(st_h.at[slot1], s1v)
    pltpu.sync_copy(lo_h, lov)
    pltpu.sync_copy(invw_h, invwv)
    pltpu.sync_copy(q0_h, qb0)
    pltpu.sync_copy(q1_h, qb1)

    c0 = c0v[...]
    s0 = s0v[...]
    c1 = c1v[...]
    s1 = s1v[...]
    lo = lov[...]
    invw = invwv[...]

    zeros16 = jnp.zeros((16,), jnp.int32)

    def zb(i, _):
        hist0[pl.ds(i * 16, 16)] = zeros16
        hist1[pl.ds(i * 16, 16)] = zeros16
        return 0

    lax.fori_loop(0, _K // 16, zb, 0)

    ones = jnp.ones((16,), jnp.int32)
    nones = jnp.full((16,), -1, jnp.int32)
    kmax = jnp.full((16,), float(_K - 1), jnp.float32)
    zf = jnp.zeros((16,), jnp.float32)

    def binof(v):
        t = (v - lo) * invw
        t = jnp.minimum(jnp.maximum(t, zf), kmax)
        return t.astype(jnp.int32)

    def scat(p0, p1):
        m = (p0 + p1) * 0.5
        plsc.addupdate_scatter(hist0, [binof(p0 * c0 + p1 * s0)], ones)
        plsc.addupdate_scatter(hist0, [binof(m * (c0 + s0))], nones)
        plsc.addupdate_scatter(hist1, [binof(p0 * c1 + p1 * s1)], ones)
        plsc.addupdate_scatter(hist1, [binof(m * (c1 + s1))], nones)

    def chunk_body(ci, _):
        pltpu.sync_copy(pd0_h.at[pl.ds(ci * _CH, _CH)], buf0)
        pltpu.sync_copy(pd1_h.at[pl.ds(ci * _CH, _CH)], buf1)

        def vbody(vi, _):
            scat(buf0[pl.ds(vi * 16, 16)], buf1[pl.ds(vi * 16, 16)])
            return 0

        lax.fori_loop(0, _CH // 16, vbody, 0)
        return 0

    lax.fori_loop(0, _NCH, chunk_body, 0)

    def qbody(vi, _):
        p0 = qb0[pl.ds(vi * 16, 16)]
        p1 = qb1[pl.ds(vi * 16, 16)]
        m = (p0 + p1) * 0.5
        plsc.addupdate_scatter(hist0, [binof(m * (c0 + s0))], ones)
        plsc.addupdate_scatter(hist0, [binof(p0 * c0 + p1 * s0)], nones)
        plsc.addupdate_scatter(hist1, [binof(m * (c1 + s1))], ones)
        plsc.addupdate_scatter(hist1, [binof(p0 * c1 + p1 * s1)], nones)
        return 0

    lax.fori_loop(0, _NQ // 16, qbody, 0)

    def absum(hist):
        def body(i, carry_acc):
            carry, acc = carry_acc
            v = hist[pl.ds(i * 16, 16)]
            cum = plsc.cumsum(v) + carry
            acc = acc + jnp.sum(jnp.abs(cum).astype(jnp.float32))
            carry = carry + jnp.sum(v)
            return (carry, acc)

        _, acc = lax.fori_loop(0, _K // 16, body, (jnp.int32(0), jnp.float32(0.0)))
        return acc

    acc0 = absum(hist0)
    acc1 = absum(hist1)
    acc1 = jnp.where(wid < 18, acc1, 0.0)

    res[...] = jnp.broadcast_to(acc0, (16,))
    pltpu.sync_copy(res, out_h.at[slot0])
    res[...] = jnp.broadcast_to(acc1, (16,))
    pltpu.sync_copy(res, out_h.at[slot1])


def _sw_pallas(pd0, pd1, q0, q1, ct, st, lo, invw):
    f = pl.kernel(
        _sw_body,
        mesh=plsc.VectorSubcoreMesh(**_SC_MESH),
        compiler_params=pltpu.CompilerParams(needs_layout_passes=False),
        out_type=jax.ShapeDtypeStruct((_NSLOT, 16), jnp.float32),
        scratch_types=[
            pltpu.VMEM((_K,), jnp.int32),
            pltpu.VMEM((_K,), jnp.int32),
            pltpu.VMEM((_CH,), jnp.float32),
            pltpu.VMEM((_CH,), jnp.float32),
            pltpu.VMEM((_NQ,), jnp.float32),
            pltpu.VMEM((_NQ,), jnp.float32),
            pltpu.VMEM((16,), jnp.float32),
            pltpu.VMEM((16,), jnp.float32),
            pltpu.VMEM((16,), jnp.float32),
            pltpu.VMEM((16,), jnp.float32),
            pltpu.VMEM((16,), jnp.float32),
            pltpu.VMEM((16,), jnp.float32),
            pltpu.VMEM((16,), jnp.float32),
        ],
    )
    return f(pd0, pd1, q0, q1, ct, st, lo, invw)


def _prelu(x):
    return jnp.where(x > 0, x, 0.1 * x)


def kernel(x0, edge_index0, PD, W1, b1, W2, b2, W4, b4, W3, b3, W5, b5, W6, b6):
    n = x0.shape[0]
    src = edge_index0[0]
    dst = edge_index0[1]

    agg = jax.ops.segment_sum(x0[src], dst, num_segments=n)
    h = jnp.maximum((x0 + agg) @ W1 + b1, 0.0)

    for W, b, relu in ((W2, b2, True), (W4, b4, True), (W3, b3, False)):
        hp = jnp.pad(h, ((0, 0), (0, _PW - _HID)))
        gathered = _g1(hp, src)[:, :_HID]
        agg = jax.ops.segment_sum(gathered, dst, num_segments=n)
        h = (h + agg) @ W + b
        if relu:
            h = jnp.maximum(h, 0.0)

    rs = src[:-n]
    rd = dst[:-n]
    a_tab = jnp.pad(h @ W5[:_HID] + b5, ((0, 0), (0, _PW - _HID)))
    b_tab = jnp.pad(h @ W5[_HID:], ((0, 0), (0, _PW - _HID)))
    ea, eb = _g2(a_tab, b_tab, rs, rd)
    pd, pd0, pd1, img, xd_sum, maxr = _pd_pallas(ea, eb, W6, b6.reshape(1, 2))

    loss_xd = (xd_sum / pd.shape[0]) / jnp.sqrt(2.0)
    loss_yd = jnp.mean(jnp.abs(PD[:, 1] - PD[:, 0])) / jnp.sqrt(2.0)

    th = np.linspace(-np.pi / 2, np.pi / 2, M_SLICES, endpoint=False)
    ct = np.zeros((_NSLOT, 16), np.float32)
    st = np.zeros((_NSLOT, 16), np.float32)
    ct[:M_SLICES, :] = np.cos(th).astype(np.float32)[:, None]
    st[:M_SLICES, :] = np.sin(th).astype(np.float32)[:, None]
    ct = jnp.asarray(ct)
    st = jnp.asarray(st)

    maxr = jnp.maximum(maxr, 2.0) + 1.0
    lo_s = -maxr
    w_s = (2.0 * maxr) / _K
    invw_s = 1.0 / w_s
    lo_v = jnp.full((16,), lo_s, jnp.float32)
    invw_v = jnp.full((16,), invw_s, jnp.float32)

    out = _sw_pallas(pd0, pd1, PD[:, 0], PD[:, 1], ct, st, lo_v, invw_v)
    n_tot = _NE + _NQ
    loss0 = jnp.sum(out[:, 0]) * w_s / (n_tot * M_SLICES)
    loss_xy = loss0

    return pd, img, loss0, loss_xy, loss_xd, loss_yd


# speedup vs baseline: 8.3168x; 1.0063x over previous
import functools

import jax
import jax.numpy as jnp
import numpy as np
from jax import lax
from jax.experimental import pallas as pl
from jax.experimental.pallas import tpu as pltpu
from jax.experimental.pallas import tpu_sc as plsc

RES = 5
SIGMA = 0.1
M_SLICES = 50

_NE = 1600000
_NA = 1700000
_NN = 100000
_HID = 32
_NQ = 1024
_K = 32768
_CH = 8000
_NCH = _NE // _CH
_NSLOT = 64

_GC = 400
_PW = 128
_E_BLOCK = 6400
_SC_MESH = dict(core_axis_name="c", subcore_axis_name="s")
_SC_PARAMS = None


def _wid():
    return lax.axis_index("s") * 2 + lax.axis_index("c")


_GW = 128


def _g1_body(t_h, src_h, g_h, idxb, buf, sem):
    wid = _wid()
    nfull = (_NA // _GC) // 32
    nrem = (_NA // _GC) - nfull * 32
    nch = jnp.where(wid < nrem, nfull + 1, nfull)

    def chunk(j, _):
        c = j * 32 + wid
        off = c * _GC
        pltpu.sync_copy(src_h.at[pl.ds(off, _GC)], idxb)
        pltpu.async_copy(t_h.at[idxb], buf, sem).wait()
        pltpu.sync_copy(buf, g_h.at[pl.ds(off, _GC)])
        return 0

    lax.fori_loop(0, nch, chunk, 0)


def _g1(table, src):
    f = pl.kernel(
        _g1_body,
        mesh=plsc.VectorSubcoreMesh(**_SC_MESH),
        compiler_params=pltpu.CompilerParams(needs_layout_passes=False),
        out_type=jax.ShapeDtypeStruct((_NA, _GW), jnp.float32),
        scratch_types=[
            pltpu.VMEM((_GC,), jnp.int32),
            pltpu.VMEM((_GC, _GW), jnp.float32),
            pltpu.SemaphoreType.DMA,
        ],
    )
    return f(table, src)


def _g2_body(a_h, b_h, rs_h, rd_h, ea_h, eb_h, idx1, idx2, bufa, bufb, sem1, sem2):
    wid = _wid()
    base = wid * (_NE // 32)

    def chunk(j, _):
        off = base + j * _GC
        pltpu.sync_copy(rs_h.at[pl.ds(off, _GC)], idx1)
        pltpu.sync_copy(rd_h.at[pl.ds(off, _GC)], idx2)
        cp1 = pltpu.async_copy(a_h.at[idx1], bufa, sem1)
        cp2 = pltpu.async_copy(b_h.at[idx2], bufb, sem2)
        cp1.wait()
        cp2.wait()
        pltpu.sync_copy(bufa, ea_h.at[pl.ds(off, _GC)])
        pltpu.sync_copy(bufb, eb_h.at[pl.ds(off, _GC)])
        return 0

    lax.fori_loop(0, _NE // 32 // _GC, chunk, 0)


def _g2(a, b, rs, rd):
    f = pl.kernel(
        _g2_body,
        mesh=plsc.VectorSubcoreMesh(**_SC_MESH),
        compiler_params=pltpu.CompilerParams(needs_layout_passes=False),
        out_type=[jax.ShapeDtypeStruct((_NE, _GW), jnp.float32),
                  jax.ShapeDtypeStruct((_NE, _GW), jnp.float32)],
        scratch_types=[
            pltpu.VMEM((_GC,), jnp.int32),
            pltpu.VMEM((_GC,), jnp.int32),
            pltpu.VMEM((_GC, _GW), jnp.float32),
            pltpu.VMEM((_GC, _GW), jnp.float32),
            pltpu.SemaphoreType.DMA,
            pltpu.SemaphoreType.DMA,
        ],
    )
    return f(a, b, rs, rd)


def _pd_body(ea_ref, eb_ref, w6_ref, b6_ref, pd_ref, p0_ref, p1_ref, sums_ref, mx_ref):
    i = pl.program_id(0)

    @pl.when(i == 0)
    def _():
        sums_ref[...] = jnp.zeros_like(sums_ref)
        mx_ref[...] = jnp.zeros_like(mx_ref)

    f = ea_ref[:, :_HID] + eb_ref[:, :_HID]
    f = jnp.where(f > 0, f, 0.1 * f)
    pdb = jnp.dot(f, w6_ref[...], preferred_element_type=jnp.float32) + b6_ref[...]
    pd_ref[...] = pdb
    p0_ref[...] = pdb[:, 0:1]
    p1_ref[...] = pdb[:, 1:2]

    births = pdb[:, 0]
    pers = pdb[:, 1] - pdb[:, 0]
    centers = (jax.lax.broadcasted_iota(jnp.int32, (1, RES), 1).astype(jnp.float32) + 0.5) / RES
    w = jnp.clip(pers, 0.0, 1.0)
    gb = jnp.exp(-((centers - births[:, None]) ** 2) / (2.0 * SIGMA**2))
    gp = jnp.exp(-((centers - pers[:, None]) ** 2) / (2.0 * SIGMA**2))
    img = jnp.dot((w[:, None] * gb).T, gp, preferred_element_type=jnp.float32)
    xd = jnp.sum(jnp.abs(pers))
    acc = jnp.pad(img, ((0, 3), (0, 123)))
    row = jax.lax.broadcasted_iota(jnp.int32, (8, 128), 0)
    col = jax.lax.broadcasted_iota(jnp.int32, (8, 128), 1)
    acc = acc + jnp.where((row == 7) & (col == 127), xd, 0.0)
    sums_ref[...] += acc
    r = jnp.max(jnp.abs(pdb[:, 0]) + jnp.abs(pdb[:, 1]))
    mx_ref[...] = jnp.maximum(mx_ref[...], jnp.full((8, 128), r))


def _pd_pallas(ea, eb, w6, b6):
    grid = _NE // _E_BLOCK
    pd, p0, p1, sums, mx = pl.pallas_call(
        _pd_body,
        grid=(grid,),
        in_specs=[
            pl.BlockSpec((_E_BLOCK, _GW), lambda i: (i, 0)),
            pl.BlockSpec((_E_BLOCK, _GW), lambda i: (i, 0)),
            pl.BlockSpec((_HID, 2), lambda i: (0, 0)),
            pl.BlockSpec((1, 2), lambda i: (0, 0)),
        ],
        out_specs=[
            pl.BlockSpec((_E_BLOCK, 2), lambda i: (i, 0)),
            pl.BlockSpec((_E_BLOCK, 1), lambda i: (i, 0)),
            pl.BlockSpec((_E_BLOCK, 1), lambda i: (i, 0)),
            pl.BlockSpec((8, 128), lambda i: (0, 0)),
            pl.BlockSpec((8, 128), lambda i: (0, 0)),
        ],
        out_shape=[
            jax.ShapeDtypeStruct((_NE, 2), jnp.float32),
            jax.ShapeDtypeStruct((_NE, 1), jnp.float32),
            jax.ShapeDtypeStruct((_NE, 1), jnp.float32),
            jax.ShapeDtypeStruct((8, 128), jnp.float32),
            jax.ShapeDtypeStruct((8, 128), jnp.float32),
        ],
    )(ea, eb, w6, b6)
    img = sums[:RES, :RES] / (2.0 * np.pi * SIGMA**2)
    xd_sum = sums[7, 127]
    maxr = mx[0, 0]
    return pd, p0.reshape(-1), p1.reshape(-1), img.reshape(-1), xd_sum, maxr


def _sw_body(pd0_h, pd1_h, q0_h, q1_h, ct_h, st_h, lo_h, invw_h, out_h,
             hist0, hist1, buf0, buf1, qb0, qb1,
             c0v, s0v, c1v, s1v, lov, invwv, res):
    wid = _wid()
    slot0 = wid
    slot1 = wid + 32

    pltpu.sync_copy(ct_h.at[slot0], c0v)
    pltpu.sync_copy(st_h.at[slot0], s0v)
    pltpu.sync_copy(ct_h.at[slot1], c1v)
    pltpu.sync_copy(st_h.at[slot1], s1v)
    pltpu.sync_copy(lo_h, lov)
    pltpu.sync_copy(invw_h, invwv)
    pltpu.sync_copy(q0_h, qb0)
    pltpu.sync_copy(q1_h, qb1)

    c0 = c0v[...]
    s0 = s0v[...]
    c1 = c1v[...]
    s1 = s1v[...]
    lo = lov[...]
    invw = invwv[...]

    zeros16 = jnp.zeros((16,), jnp.int32)

    def zb(i, _):
        hist0[pl.ds(i * 16, 16)] = zeros16
        hist1[pl.ds(i * 16, 16)] = zeros16
        return 0

    lax.fori_loop(0, _K // 16, zb, 0)

    ones = jnp.ones((16,), jnp.int32)
    nones = jnp.full((16,), -1, jnp.int32)
    kmax = jnp.full((16,), float(_K - 1), jnp.float32)
    zf = jnp.zeros((16,), jnp.float32)

    def binof(v):
        t = (v - lo) * invw
        t = jnp.minimum(jnp.maximum(t, zf), kmax)
        return t.astype(jnp.int32)

    def scat(p0, p1):
        m = (p0 + p1) * 0.5
        plsc.addupdate_scatter(hist0, [binof(p0 * c0 + p1 * s0)], ones)
        plsc.addupdate_scatter(hist0, [binof(m * (c0 + s0))], nones)
        plsc.addupdate_scatter(hist1, [binof(p0 * c1 + p1 * s1)], ones)
        plsc.addupdate_scatter(hist1, [binof(m * (c1 + s1))], nones)

    def chunk_body(ci, _):
        pltpu.sync_copy(pd0_h.at[pl.ds(ci * _CH, _CH)], buf0)
        pltpu.sync_copy(pd1_h.at[pl.ds(ci * _CH, _CH)], buf1)

        def vbody(vi, _):
            scat(buf0[pl.ds(vi * 16, 16)], buf1[pl.ds(vi * 16, 16)])
            return 0

        lax.fori_loop(0, _CH // 16, vbody, 0)
        return 0

    lax.fori_loop(0, _NCH, chunk_body, 0)

    def qbody(vi, _):
        p0 = qb0[pl.ds(vi * 16, 16)]
        p1 = qb1[pl.ds(vi * 16, 16)]
        m = (p0 + p1) * 0.5
        plsc.addupdate_scatter(hist0, [binof(m * (c0 + s0))], ones)
        plsc.addupdate_scatter(hist0, [binof(p0 * c0 + p1 * s0)], nones)
        plsc.addupdate_scatter(hist1, [binof(m * (c1 + s1))], ones)
        plsc.addupdate_scatter(hist1, [binof(p0 * c1 + p1 * s1)], nones)
        return 0

    lax.fori_loop(0, _NQ // 16, qbody, 0)

    def absum(hist):
        def body(i, carry_acc):
            carry, acc = carry_acc
            v = hist[pl.ds(i * 16, 16)]
            cum = plsc.cumsum(v) + carry
            acc = acc + jnp.sum(jnp.abs(cum).astype(jnp.float32))
            carry = carry + jnp.sum(v)
            return (carry, acc)

        _, acc = lax.fori_loop(0, _K // 16, body, (jnp.int32(0), jnp.float32(0.0)))
        return acc

    acc0 = absum(hist0)
    acc1 = absum(hist1)
    acc1 = jnp.where(wid < 18, acc1, 0.0)

    res[...] = jnp.broadcast_to(acc0, (16,))
    pltpu.sync_copy(res, out_h.at[slot0])
    res[...] = jnp.broadcast_to(acc1, (16,))
    pltpu.sync_copy(res, out_h.at[slot1])


def _sw_pallas(pd0, pd1, q0, q1, ct, st, lo, invw):
    f = pl.kernel(
        _sw_body,
        mesh=plsc.VectorSubcoreMesh(**_SC_MESH),
        compiler_params=pltpu.CompilerParams(needs_layout_passes=False),
        out_type=jax.ShapeDtypeStruct((_NSLOT, 16), jnp.float32),
        scratch_types=[
            pltpu.VMEM((_K,), jnp.int32),
            pltpu.VMEM((_K,), jnp.int32),
            pltpu.VMEM((_CH,), jnp.float32),
            pltpu.VMEM((_CH,), jnp.float32),
            pltpu.VMEM((_NQ,), jnp.float32),
            pltpu.VMEM((_NQ,), jnp.float32),
            pltpu.VMEM((16,), jnp.float32),
            pltpu.VMEM((16,), jnp.float32),
            pltpu.VMEM((16,), jnp.float32),
            pltpu.VMEM((16,), jnp.float32),
            pltpu.VMEM((16,), jnp.float32),
            pltpu.VMEM((16,), jnp.float32),
            pltpu.VMEM((16,), jnp.float32),
        ],
    )
    return f(pd0, pd1, q0, q1, ct, st, lo, invw)


def _prelu(x):
    return jnp.where(x > 0, x, 0.1 * x)


def kernel(x0, edge_index0, PD, W1, b1, W2, b2, W4, b4, W3, b3, W5, b5, W6, b6):
    n = x0.shape[0]
    src = edge_index0[0]
    dst = edge_index0[1]

    agg = jax.ops.segment_sum(x0[src], dst, num_segments=n)
    h = jnp.maximum((x0 + agg) @ W1 + b1, 0.0)

    for W, b, relu in ((W2, b2, True), (W4, b4, True), (W3, b3, False)):
        hp = jnp.pad(h, ((0, 0), (0, _GW - _HID)))
        gathered = _g1(hp, src)[:, :_HID]
        agg = jax.ops.segment_sum(gathered, dst, num_segments=n)
        h = (h + agg) @ W + b
        if relu:
            h = jnp.maximum(h, 0.0)

    rs = src[:-n]
    rd = dst[:-n]
    a_tab = jnp.pad(h @ W5[:_HID] + b5, ((0, 0), (0, _GW - _HID)))
    b_tab = jnp.pad(h @ W5[_HID:], ((0, 0), (0, _GW - _HID)))
    ea, eb = _g2(a_tab, b_tab, rs, rd)
    pd, pd0, pd1, img, xd_sum, maxr = _pd_pallas(ea, eb, W6, b6.reshape(1, 2))

    loss_xd = (xd_sum / pd.shape[0]) / jnp.sqrt(2.0)
    loss_yd = jnp.mean(jnp.abs(PD[:, 1] - PD[:, 0])) / jnp.sqrt(2.0)

    th = np.linspace(-np.pi / 2, np.pi / 2, M_SLICES, endpoint=False)
    ct = np.zeros((_NSLOT, 16), np.float32)
    st = np.zeros((_NSLOT, 16), np.float32)
    ct[:M_SLICES, :] = np.cos(th).astype(np.float32)[:, None]
    st[:M_SLICES, :] = np.sin(th).astype(np.float32)[:, None]
    ct = jnp.asarray(ct)
    st = jnp.asarray(st)

    maxr = jnp.maximum(maxr, 2.0) + 1.0
    lo_s = -maxr
    w_s = (2.0 * maxr) / _K
    invw_s = 1.0 / w_s
    lo_v = jnp.full((16,), lo_s, jnp.float32)
    invw_v = jnp.full((16,), invw_s, jnp.float32)

    out = _sw_pallas(pd0, pd1, PD[:, 0], PD[:, 1], ct, st, lo_v, invw_v)
    n_tot = _NE + _NQ
    loss0 = jnp.sum(out[:, 0]) * w_s / (n_tot * M_SLICES)
    loss_xy = loss0

    return pd, img, loss0, loss_xy, loss_xd, loss_yd
